# baseline (speedup 1.0000x reference)
import jax
import jax.numpy as jnp
from jax.experimental import pallas as pl
from jax.experimental.pallas import tpu as pltpu

L = 4096
H = 64
W = 64
DM = 96
DI = 192
NS = 16
RK = 6
K = 4
REP = 64
CH = 64
NCH = L // CH


def _pre_kernel(x_ref, repf_ref, win_ref, w9_ref, sb_ref, smw_ref, smb_ref,
                alw_ref, alb_ref, dsw_ref, dsb_ref, gw_ref, gb_ref, bw_ref,
                bb_ref, xc_ref, arow_ref, dsk_ref, gam_ref, bet_ref, pad_ref):
    f32 = jnp.float32
    repg = jnp.mean(repf_ref[...], axis=0, keepdims=True)
    dot = lambda a, b: jax.lax.dot_general(
        a, b, (((1,), (0,)), ((), ())), preferred_element_type=f32)
    s = 1.0 + dot(repg, smw_ref[...]) + smb_ref[...]
    arow_ref[...] = -jnp.exp(dot(repg, alw_ref[...]) + alb_ref[...])
    dsk_ref[...] = dot(repg, dsw_ref[...]) + dsb_ref[...]
    gam_ref[...] = dot(repg, gw_ref[...]) + gb_ref[...]
    bet_ref[...] = dot(repg, bw_ref[...]) + bb_ref[...]
    xi = dot(x_ref[...], win_ref[...]) * s
    pad_ref[...] = jnp.zeros_like(pad_ref)
    pad_ref[1:H + 1, 1:W + 1, :] = xi.reshape(H, W, DI)
    acc = jnp.zeros((H, W, DI), f32)
    for i in range(3):
        for j in range(3):
            wt = w9_ref[i * 3 + j:i * 3 + j + 1, :].reshape(1, 1, DI)
            acc = acc + wt * pad_ref[i:i + H, j:j + W, :]
    acc = acc + sb_ref[...].reshape(1, 1, DI)
    xc = acc * jax.lax.logistic(acc)
    xc_ref[...] = xc.reshape(L, DI)


def _scan_kernel(x0f_ref, x0b_ref, x1f_ref, x1b_ref, w76_ref, dtw_ref,
                 dtb_ref, a_ref, dsk_ref, pt_ref, sel_ref, bones_ref,
                 r16_ref, y0f_ref, y0b_ref, y1f_ref, y1b_ref,
                 at_s, da_s, bu_s, cb_s, ha_s, h_s):
    f32 = jnp.float32
    c = pl.program_id(0)

    @pl.when(c == 0)
    def _():
        h_s[...] = jnp.zeros_like(h_s)
        for d in range(4):
            at_s[d] = jnp.broadcast_to(
                a_ref[d // 2, d % 2].reshape(1, NS, DI),
                (CH, NS, DI)).reshape(CH * NS, DI)

    dt = (((1,), (1,)), ((), ()))
    sel = sel_ref[...]
    bones = bones_ref[...]
    r16 = r16_ref[...]
    xv4 = [x0f_ref[...], x0b_ref[...],
           x1f_ref[...].reshape(CH, DI), x1b_ref[...].reshape(CH, DI)]
    for p in range(2):
        xf = xv4[2 * p]
        xb = xv4[2 * p + 1]
        w76 = w76_ref[p]
        xdbl_f = jax.lax.dot_general(xf, w76, dt, preferred_element_type=f32)
        xdbl_b = jax.lax.dot_general(xb, w76, dt, preferred_element_type=f32)
        dts = jnp.concatenate([xdbl_f[:, 0:RK], xdbl_b[:, 38:38 + RK]],
                              axis=1)
        draw = jax.lax.dot_general(dts, dtw_ref[p], (((1,), (0,)), ((), ())),
                                   preferred_element_type=f32) + dtb_ref[p]
        q = jnp.maximum(draw, 0.0) \
            + jnp.log2(1.0 + jnp.exp2(-jnp.abs(draw)))
        ucat = q * jnp.concatenate([xf, xb], axis=1)
        cat = jnp.concatenate(
            [q, ucat, xdbl_f[:, RK:RK + 2 * NS], xdbl_b[:, 38 + RK:76]],
            axis=1)
        rep = jax.lax.dot_general(r16, cat, (((1,), (0,)), ((), ())),
                                  preferred_element_type=f32)
        for fb in range(2):
            d = 2 * p + fb
            da_s[d] = jnp.exp2(rep[:, fb * DI:(fb + 1) * DI] * at_s[d])
            bct = rep[:, 768 + 32 * fb:800 + 32 * fb]
            bbcc = jax.lax.dot_general(bct * sel, bones,
                                       (((1,), (0,)), ((), ())),
                                       preferred_element_type=f32)
            bu_s[d] = bbcc[:, 0:DI] \
                * rep[:, 384 + fb * DI:384 + (fb + 1) * DI]
            cb_s[d] = bbcc[:, DI:2 * DI]

    hs = [h_s[i] for i in range(4)]
    for s in range(CH):
        rows = (s, CH - 1 - s, s, CH - 1 - s)
        for d in range(4):
            t = rows[d]
            hs[d] = da_s[d, NS * t:NS * t + NS, :] * hs[d] \
                + bu_s[d, NS * t:NS * t + NS, :]
            ha_s[d, NS * t:NS * t + NS, :] = hs[d]
    for d in range(4):
        h_s[d] = hs[d]

    dsks = [dsk_ref[0, 0:1, :], dsk_ref[0, 1:2, :],
            dsk_ref[1, 0:1, :], dsk_ref[1, 1:2, :]]
    yrefs = (y0f_ref, y0b_ref, y1f_ref, y1b_ref)
    pt = pt_ref[...]
    for d in range(4):
        prod = ha_s[d] * cb_s[d]
        y2 = jax.lax.dot_general(pt, prod, (((1,), (0,)), ((), ())),
                                 preferred_element_type=f32)
        yv = y2 + dsks[d] * xv4[d]
        if d < 2:
            yrefs[d][...] = yv
        else:
            yrefs[d][...] = yv.reshape(CH, 1, 1, DI)


def _fin_kernel(p0_ref, p1_ref, p2_ref, p3_ref, gam_ref, bet_ref, wout_ref,
                o_ref):
    y = p0_ref[...] + p1_ref[...] + p2_ref[...] + p3_ref[...]
    mu = jnp.mean(y, axis=0, keepdims=True)
    d = y - mu
    var = jnp.mean(d * d, axis=0, keepdims=True)
    yn = d * jax.lax.rsqrt(var + 1e-5)
    z = yn * gam_ref[...] + bet_ref[...]
    o_ref[...] = jnp.dot(z, wout_ref[...], preferred_element_type=jnp.float32)


def kernel(x, rep, in_proj_w, sconv_w, sconv_b, sconv_mod_w, sconv_mod_b,
           x_proj_weight, dt_projs_weight, dt_projs_bias, A_logs_w, A_logs_b,
           Ds_w, Ds_b, sain_gamma_w, sain_gamma_b, sain_beta_w, sain_beta_b,
           out_proj_w):
    f32 = jnp.float32
    sds = jax.ShapeDtypeStruct
    xflat = x.reshape(L, DM)
    repf = rep.reshape(REP, L).T

    xc, arow, dskrow, gam, bet = pl.pallas_call(
        _pre_kernel,
        out_shape=[sds((L, DI), f32), sds((1, K * DI * NS), f32),
                   sds((1, K * DI), f32), sds((1, DI), f32), sds((1, DI), f32)],
        scratch_shapes=[pltpu.VMEM((H + 2, W + 2, DI), f32)],
        compiler_params=pltpu.CompilerParams(vmem_limit_bytes=56 * 2**20),
        name="ss2d_pre",
    )(xflat, repf, in_proj_w.T, sconv_w.reshape(DI, 9).T,
      sconv_b[None, :], sconv_mod_w.T, sconv_mod_b[None, :],
      A_logs_w.T, A_logs_b[None, :], Ds_w.T, Ds_b[None, :],
      sain_gamma_w.T, sain_gamma_b[None, :], sain_beta_w.T,
      sain_beta_b[None, :])

    xc4 = xc.reshape(H, W, 1, DI)
    A_T = arow.reshape(K, DI, NS).transpose(0, 2, 1)
    A_pair = jnp.stack([jnp.stack([A_T[0], A_T[2]]),
                        jnp.stack([A_T[1], A_T[3]])])
    dsk = dskrow.reshape(K, DI)
    dsk_pair = jnp.stack([jnp.stack([dsk[0], dsk[2]]),
                          jnp.stack([dsk[1], dsk[3]])])
    wx = x_proj_weight
    w76 = jnp.stack([jnp.concatenate([wx[0], wx[2]], axis=0),
                     jnp.concatenate([wx[1], wx[3]], axis=0)])
    log2e = jnp.float32(1.4426950408889634)
    z6 = jnp.zeros((RK, DI), f32)
    dtw = dt_projs_weight
    dtw_pair = jnp.stack([
        jnp.concatenate([jnp.concatenate([dtw[0].T, z6], axis=1),
                         jnp.concatenate([z6, dtw[2].T], axis=1)], axis=0),
        jnp.concatenate([jnp.concatenate([dtw[1].T, z6], axis=1),
                         jnp.concatenate([z6, dtw[3].T], axis=1)], axis=0),
    ]) * log2e
    dtb = dt_projs_bias
    dtb_pair = jnp.stack([
        jnp.concatenate([dtb[0], dtb[2]])[None, :],
        jnp.concatenate([dtb[1], dtb[3]])[None, :],
    ]) * log2e

    blk = lambda im: pl.BlockSpec((CH, DI), im)
    blk4 = lambda im: pl.BlockSpec((H, 1, 1, DI), im)
    fwd = lambda c: (c, 0)
    bwd = lambda c: (NCH - 1 - c, 0)
    fwd4 = lambda c: (0, c, 0, 0)
    bwd4 = lambda c: (0, NCH - 1 - c, 0, 0)
    ri = jnp.arange(CH * NS, dtype=jnp.int32)[:, None] % NS
    ci = jnp.arange(2 * NS, dtype=jnp.int32)[None, :] % NS
    sel = (ri == ci).astype(f32)
    br = jnp.arange(2 * NS)[:, None] < NS
    bc = jnp.arange(2 * DI)[None, :] < DI
    bones = jnp.where(br & bc, jnp.float32(0.6931471805599453), 0.0) \
        + ((~br) & (~bc)).astype(f32)
    y0f, y0b, y1f, y1b = pl.pallas_call(
        _scan_kernel,
        grid=(NCH,),
        in_specs=[
            blk(fwd), blk(bwd), blk4(fwd4), blk4(bwd4),
            pl.BlockSpec((2, 76, DI), lambda c: (0, 0, 0)),
            pl.BlockSpec((2, 2 * RK, 2 * DI), lambda c: (0, 0, 0)),
            pl.BlockSpec((2, 1, 2 * DI), lambda c: (0, 0, 0)),
            pl.BlockSpec((2, 2, NS, DI), lambda c: (0, 0, 0, 0)),
            pl.BlockSpec((2, 2, DI), lambda c: (0, 0, 0)),
            pl.BlockSpec((CH, CH * NS), lambda c: (0, 0)),
            pl.BlockSpec((CH * NS, 2 * NS), lambda c: (0, 0)),
            pl.BlockSpec((2 * NS, 2 * DI), lambda c: (0, 0)),
            pl.BlockSpec((CH * NS, CH), lambda c: (0, 0)),
        ],
        out_specs=[blk(fwd), blk(bwd), blk4(fwd4), blk4(bwd4)],
        out_shape=[sds((L, DI), f32), sds((L, DI), f32),
                   sds((H, W, 1, DI), f32), sds((H, W, 1, DI), f32)],
        scratch_shapes=[
            pltpu.VMEM((4, CH * NS, DI), f32),
            pltpu.VMEM((4, CH * NS, DI), f32),
            pltpu.VMEM((4, CH * NS, DI), f32),
            pltpu.VMEM((4, CH * NS, DI), f32),
            pltpu.VMEM((4, CH * NS, DI), f32),
            pltpu.VMEM((4, NS, DI), f32),
        ],
        compiler_params=pltpu.CompilerParams(
            dimension_semantics=(pltpu.GridDimensionSemantics.ARBITRARY,),
            vmem_limit_bytes=40 * 2**20,
        ),
        name="ss2d_scan",
    )(xc, xc, xc4, xc4, w76, dtw_pair, dtb_pair, A_pair, dsk_pair,
      jnp.repeat(jnp.eye(CH, dtype=f32), NS, axis=1), sel, bones,
      jnp.repeat(jnp.eye(CH, dtype=f32), NS, axis=0))

    p2 = y1f.reshape(L, DI)
    p3 = y1b.reshape(L, DI)
    out = pl.pallas_call(
        _fin_kernel,
        out_shape=sds((L, DM), f32),
        compiler_params=pltpu.CompilerParams(vmem_limit_bytes=56 * 2**20),
        name="ss2d_fin",
    )(y0f, y0b, p2, p3, gam, bet, out_proj_w.T)
    return out.reshape(1, H, W, DM)


# speedup vs baseline: 1.0982x; 1.0982x over previous
import jax
import jax.numpy as jnp
from jax.experimental import pallas as pl
from jax.experimental.pallas import tpu as pltpu

L = 4096
H = 64
W = 64
DM = 96
DI = 192
NS = 16
RK = 6
K = 4
REP = 64
CH = 128
NCH = L // CH


def _pre_kernel(x_ref, repf_ref, win_ref, w9_ref, sb_ref, smw_ref, smb_ref,
                alw_ref, alb_ref, dsw_ref, dsb_ref, gw_ref, gb_ref, bw_ref,
                bb_ref, xc_ref, arow_ref, dsk_ref, gam_ref, bet_ref, pad_ref):
    f32 = jnp.float32
    repg = jnp.mean(repf_ref[...], axis=0, keepdims=True)
    dot = lambda a, b: jax.lax.dot_general(
        a, b, (((1,), (0,)), ((), ())), preferred_element_type=f32)
    s = 1.0 + dot(repg, smw_ref[...]) + smb_ref[...]
    arow_ref[...] = -jnp.exp(dot(repg, alw_ref[...]) + alb_ref[...])
    dsk_ref[...] = dot(repg, dsw_ref[...]) + dsb_ref[...]
    gam_ref[...] = dot(repg, gw_ref[...]) + gb_ref[...]
    bet_ref[...] = dot(repg, bw_ref[...]) + bb_ref[...]
    xi = dot(x_ref[...], win_ref[...]) * s
    pad_ref[...] = jnp.zeros_like(pad_ref)
    pad_ref[1:H + 1, 1:W + 1, :] = xi.reshape(H, W, DI)
    acc = jnp.zeros((H, W, DI), f32)
    for i in range(3):
        for j in range(3):
            wt = w9_ref[i * 3 + j:i * 3 + j + 1, :].reshape(1, 1, DI)
            acc = acc + wt * pad_ref[i:i + H, j:j + W, :]
    acc = acc + sb_ref[...].reshape(1, 1, DI)
    xc = acc * jax.lax.logistic(acc)
    xc_ref[...] = xc.reshape(L, DI)


def _scan_kernel(x0f_ref, x0b_ref, x1f_ref, x1b_ref, w76_ref, dtw_ref,
                 dtb_ref, a_ref, dsk_ref, pt_ref, sel_ref, bones_ref,
                 y0f_ref, y0b_ref, y1f_ref, y1b_ref,
                 da_s, bu_s, cb_s, ha_s, h_s):
    f32 = jnp.float32
    c = pl.program_id(0)

    @pl.when(c == 0)
    def _():
        h_s[...] = jnp.zeros_like(h_s)

    dt = (((1,), (1,)), ((), ()))
    sel = sel_ref[...]
    bones = bones_ref[...]
    xv4 = [x0f_ref[...], x0b_ref[...],
           jnp.concatenate([x1f_ref[:, 0, 0, :], x1f_ref[:, 1, 0, :]],
                           axis=0),
           jnp.concatenate([x1b_ref[:, 0, 0, :], x1b_ref[:, 1, 0, :]],
                           axis=0)]
    for p in range(2):
        xf = xv4[2 * p]
        xb = xv4[2 * p + 1]
        w76 = w76_ref[p]
        xdbl_f = jax.lax.dot_general(xf, w76, dt, preferred_element_type=f32)
        xdbl_b = jax.lax.dot_general(xb, w76, dt, preferred_element_type=f32)
        dts = jnp.concatenate([xdbl_f[:, 0:RK], xdbl_b[:, 38:38 + RK]],
                              axis=1)
        draw = jax.lax.dot_general(dts, dtw_ref[p], (((1,), (0,)), ((), ())),
                                   preferred_element_type=f32) + dtb_ref[p]
        q = jnp.maximum(draw, 0.0) \
            + jnp.log2(1.0 + jnp.exp2(-jnp.abs(draw)))
        for fb in range(2):
            d = 2 * p + fb
            xv = xf if fb == 0 else xb
            xdbl = xdbl_f if fb == 0 else xdbl_b
            qd = q[:, fb * DI:(fb + 1) * DI]
            da_s[d] = jnp.exp2(qd.reshape(CH, 1, DI)
                               * a_ref[p, fb].reshape(1, NS, DI))
            u3 = (qd * xv).reshape(CH, 1, DI)
            bc2 = xdbl[:, fb * 38 + RK:fb * 38 + RK + 2 * NS]
            bct = jnp.broadcast_to(bc2.reshape(CH, 1, 2 * NS),
                                   (CH, NS, 2 * NS)).reshape(CH * NS, 2 * NS)
            bbcc = jax.lax.dot_general(bct * sel, bones,
                                       (((1,), (0,)), ((), ())),
                                       preferred_element_type=f32)
            bu_s[d] = (bbcc[:, 0:DI].reshape(CH, NS, DI) * u3)
            cb_s[d] = bbcc[:, DI:2 * DI].reshape(CH, NS, DI)

    hs = [h_s[i] for i in range(4)]
    for s in range(CH):
        rows = (s, CH - 1 - s, s, CH - 1 - s)
        for d in range(4):
            t = rows[d]
            hs[d] = da_s[d, t] * hs[d] + bu_s[d, t]
            ha_s[d, t] = hs[d]
    for d in range(4):
        h_s[d] = hs[d]

    dsks = [dsk_ref[0, 0:1, :], dsk_ref[0, 1:2, :],
            dsk_ref[1, 0:1, :], dsk_ref[1, 1:2, :]]
    yrefs = (y0f_ref, y0b_ref, y1f_ref, y1b_ref)
    pt = pt_ref[...]
    for d in range(4):
        prod = (ha_s[d] * cb_s[d]).reshape(CH * NS, DI)
        y2 = jax.lax.dot_general(pt, prod, (((1,), (0,)), ((), ())),
                                 preferred_element_type=f32)
        yv = y2 + dsks[d] * xv4[d]
        if d < 2:
            yrefs[d][...] = yv
        else:
            yrefs[d][:, 0:1, 0:1, :] = yv[0:H].reshape(H, 1, 1, DI)
            yrefs[d][:, 1:2, 0:1, :] = yv[H:CH].reshape(H, 1, 1, DI)


def _fin_kernel(p0_ref, p1_ref, p2_ref, p3_ref, gam_ref, bet_ref, wout_ref,
                o_ref):
    y = p0_ref[...] + p1_ref[...] + p2_ref[...] + p3_ref[...]
    mu = jnp.mean(y, axis=0, keepdims=True)
    d = y - mu
    var = jnp.mean(d * d, axis=0, keepdims=True)
    yn = d * jax.lax.rsqrt(var + 1e-5)
    z = yn * gam_ref[...] + bet_ref[...]
    o_ref[...] = jnp.dot(z, wout_ref[...], preferred_element_type=jnp.float32)


def kernel(x, rep, in_proj_w, sconv_w, sconv_b, sconv_mod_w, sconv_mod_b,
           x_proj_weight, dt_projs_weight, dt_projs_bias, A_logs_w, A_logs_b,
           Ds_w, Ds_b, sain_gamma_w, sain_gamma_b, sain_beta_w, sain_beta_b,
           out_proj_w):
    f32 = jnp.float32
    sds = jax.ShapeDtypeStruct
    xflat = x.reshape(L, DM)
    repf = rep.reshape(REP, L).T

    xc, arow, dskrow, gam, bet = pl.pallas_call(
        _pre_kernel,
        out_shape=[sds((L, DI), f32), sds((1, K * DI * NS), f32),
                   sds((1, K * DI), f32), sds((1, DI), f32), sds((1, DI), f32)],
        scratch_shapes=[pltpu.VMEM((H + 2, W + 2, DI), f32)],
        compiler_params=pltpu.CompilerParams(vmem_limit_bytes=56 * 2**20),
        name="ss2d_pre",
    )(xflat, repf, in_proj_w.T, sconv_w.reshape(DI, 9).T,
      sconv_b[None, :], sconv_mod_w.T, sconv_mod_b[None, :],
      A_logs_w.T, A_logs_b[None, :], Ds_w.T, Ds_b[None, :],
      sain_gamma_w.T, sain_gamma_b[None, :], sain_beta_w.T,
      sain_beta_b[None, :])

    xc4 = xc.reshape(H, W, 1, DI)
    A_T = arow.reshape(K, DI, NS).transpose(0, 2, 1)
    A_pair = jnp.stack([jnp.stack([A_T[0], A_T[2]]),
                        jnp.stack([A_T[1], A_T[3]])])
    dsk = dskrow.reshape(K, DI)
    dsk_pair = jnp.stack([jnp.stack([dsk[0], dsk[2]]),
                          jnp.stack([dsk[1], dsk[3]])])
    wx = x_proj_weight
    w76 = jnp.stack([jnp.concatenate([wx[0], wx[2]], axis=0),
                     jnp.concatenate([wx[1], wx[3]], axis=0)])
    log2e = jnp.float32(1.4426950408889634)
    z6 = jnp.zeros((RK, DI), f32)
    dtw = dt_projs_weight
    dtw_pair = jnp.stack([
        jnp.concatenate([jnp.concatenate([dtw[0].T, z6], axis=1),
                         jnp.concatenate([z6, dtw[2].T], axis=1)], axis=0),
        jnp.concatenate([jnp.concatenate([dtw[1].T, z6], axis=1),
                         jnp.concatenate([z6, dtw[3].T], axis=1)], axis=0),
    ]) * log2e
    dtb = dt_projs_bias
    dtb_pair = jnp.stack([
        jnp.concatenate([dtb[0], dtb[2]])[None, :],
        jnp.concatenate([dtb[1], dtb[3]])[None, :],
    ]) * log2e

    blk = lambda im: pl.BlockSpec((CH, DI), im)
    blk4 = lambda im: pl.BlockSpec((H, CH // H, 1, DI), im)
    fwd = lambda c: (c, 0)
    bwd = lambda c: (NCH - 1 - c, 0)
    fwd4 = lambda c: (0, c, 0, 0)
    bwd4 = lambda c: (0, NCH - 1 - c, 0, 0)
    ri = jnp.arange(CH * NS, dtype=jnp.int32)[:, None] % NS
    ci = jnp.arange(2 * NS, dtype=jnp.int32)[None, :] % NS
    sel = (ri == ci).astype(f32)
    br = jnp.arange(2 * NS)[:, None] < NS
    bc = jnp.arange(2 * DI)[None, :] < DI
    bones = jnp.where(br & bc, jnp.float32(0.6931471805599453), 0.0) \
        + ((~br) & (~bc)).astype(f32)
    y0f, y0b, y1f, y1b = pl.pallas_call(
        _scan_kernel,
        grid=(NCH,),
        in_specs=[
            blk(fwd), blk(bwd), blk4(fwd4), blk4(bwd4),
            pl.BlockSpec((2, 76, DI), lambda c: (0, 0, 0)),
            pl.BlockSpec((2, 2 * RK, 2 * DI), lambda c: (0, 0, 0)),
            pl.BlockSpec((2, 1, 2 * DI), lambda c: (0, 0, 0)),
            pl.BlockSpec((2, 2, NS, DI), lambda c: (0, 0, 0, 0)),
            pl.BlockSpec((2, 2, DI), lambda c: (0, 0, 0)),
            pl.BlockSpec((CH, CH * NS), lambda c: (0, 0)),
            pl.BlockSpec((CH * NS, 2 * NS), lambda c: (0, 0)),
            pl.BlockSpec((2 * NS, 2 * DI), lambda c: (0, 0)),
        ],
        out_specs=[blk(fwd), blk(bwd), blk4(fwd4), blk4(bwd4)],
        out_shape=[sds((L, DI), f32), sds((L, DI), f32),
                   sds((H, W, 1, DI), f32), sds((H, W, 1, DI), f32)],
        scratch_shapes=[
            pltpu.VMEM((4, CH, NS, DI), f32),
            pltpu.VMEM((4, CH, NS, DI), f32),
            pltpu.VMEM((4, CH, NS, DI), f32),
            pltpu.VMEM((4, CH, NS, DI), f32),
            pltpu.VMEM((4, NS, DI), f32),
        ],
        compiler_params=pltpu.CompilerParams(
            dimension_semantics=(pltpu.GridDimensionSemantics.ARBITRARY,),
            vmem_limit_bytes=48 * 2**20,
        ),
        name="ss2d_scan",
    )(xc, xc, xc4, xc4, w76, dtw_pair, dtb_pair, A_pair, dsk_pair,
      jnp.repeat(jnp.eye(CH, dtype=f32), NS, axis=1), sel, bones)

    p2 = y1f.reshape(L, DI)
    p3 = y1b.reshape(L, DI)
    out = pl.pallas_call(
        _fin_kernel,
        out_shape=sds((L, DM), f32),
        compiler_params=pltpu.CompilerParams(vmem_limit_bytes=56 * 2**20),
        name="ss2d_fin",
    )(y0f, y0b, p2, p3, gam, bet, out_proj_w.T)
    return out.reshape(1, H, W, DM)


# speedup vs baseline: 1.1045x; 1.0057x over previous
import jax
import jax.numpy as jnp
from jax.experimental import pallas as pl
from jax.experimental.pallas import tpu as pltpu

L = 4096
H = 64
W = 64
DM = 96
DI = 192
NS = 16
RK = 6
K = 4
REP = 64
CH = 128
NCH = L // CH


def _pre_kernel(x_ref, repf_ref, win_ref, w9_ref, sb_ref, smw_ref, smb_ref,
                alw_ref, alb_ref, dsw_ref, dsb_ref, gw_ref, gb_ref, bw_ref,
                bb_ref, xc_ref, arow_ref, dsk_ref, gam_ref, bet_ref, pad_ref,
                sh1_ref, sh2_ref):
    f32 = jnp.float32
    repg = jnp.mean(repf_ref[...], axis=0, keepdims=True)
    dot = lambda a, b: jax.lax.dot_general(
        a, b, (((1,), (0,)), ((), ())), preferred_element_type=f32)
    s = 1.0 + dot(repg, smw_ref[...]) + smb_ref[...]
    arow_ref[...] = -jnp.exp(dot(repg, alw_ref[...]) + alb_ref[...])
    dsk_ref[...] = dot(repg, dsw_ref[...]) + dsb_ref[...]
    gam_ref[...] = dot(repg, gw_ref[...]) + gb_ref[...]
    bet_ref[...] = dot(repg, bw_ref[...]) + bb_ref[...]
    xi = dot(x_ref[...], win_ref[...]) * s
    pad_ref[...] = jnp.zeros_like(pad_ref)
    pad_ref[1:H + 1, 1:W + 1, :] = xi.reshape(H, W, DI)
    sh1_ref[...] = pad_ref[:, 1:W + 1, :]
    sh2_ref[...] = pad_ref[:, 2:W + 2, :]
    taps = (pad_ref, sh1_ref, sh2_ref)
    acc = jnp.zeros((H, W, DI), f32)
    for i in range(3):
        for j in range(3):
            wt = w9_ref[i * 3 + j:i * 3 + j + 1, :].reshape(1, 1, DI)
            acc = acc + wt * taps[j][i:i + H, 0:W, :]
    acc = acc + sb_ref[...].reshape(1, 1, DI)
    xc = acc * jax.lax.logistic(acc)
    xc_ref[...] = xc.reshape(L, DI)


def _scan_kernel(x0f_ref, x0b_ref, x1f_ref, x1b_ref, w76_ref, dtw_ref,
                 dtb_ref, a_ref, dsk_ref, pt_ref, sel_ref, bones_ref,
                 y0f_ref, y0b_ref, y1f_ref, y1b_ref,
                 da_s, bu_s, cb_s, ha_s, h_s):
    f32 = jnp.float32
    c = pl.program_id(0)

    @pl.when(c == 0)
    def _():
        h_s[...] = jnp.zeros_like(h_s)

    dt = (((1,), (1,)), ((), ()))
    sel = sel_ref[...]
    bones = bones_ref[...]
    xv4 = [x0f_ref[...], x0b_ref[...],
           jnp.concatenate([x1f_ref[:, 0, 0, :], x1f_ref[:, 1, 0, :]],
                           axis=0),
           jnp.concatenate([x1b_ref[:, 0, 0, :], x1b_ref[:, 1, 0, :]],
                           axis=0)]
    for p in range(2):
        xf = xv4[2 * p]
        xb = xv4[2 * p + 1]
        w76 = w76_ref[p]
        xdbl_f = jax.lax.dot_general(xf, w76, dt, preferred_element_type=f32)
        xdbl_b = jax.lax.dot_general(xb, w76, dt, preferred_element_type=f32)
        dts = jnp.concatenate([xdbl_f[:, 0:RK], xdbl_b[:, 38:38 + RK]],
                              axis=1)
        draw = jax.lax.dot_general(dts, dtw_ref[p], (((1,), (0,)), ((), ())),
                                   preferred_element_type=f32) + dtb_ref[p]
        q = jnp.maximum(draw, 0.0) \
            + jnp.log2(1.0 + jnp.exp2(-jnp.abs(draw)))
        for fb in range(2):
            d = 2 * p + fb
            xv = xf if fb == 0 else xb
            xdbl = xdbl_f if fb == 0 else xdbl_b
            qd = q[:, fb * DI:(fb + 1) * DI]
            da_s[d] = jnp.exp2(qd.reshape(CH, 1, DI)
                               * a_ref[p, fb].reshape(1, NS, DI))
            u3 = (qd * xv).reshape(CH, 1, DI)
            bc2 = xdbl[:, fb * 38 + RK:fb * 38 + RK + 2 * NS]
            bct = jnp.broadcast_to(bc2.reshape(CH, 1, 2 * NS),
                                   (CH, NS, 2 * NS)).reshape(CH * NS, 2 * NS)
            bbcc = jax.lax.dot_general(bct * sel, bones,
                                       (((1,), (0,)), ((), ())),
                                       preferred_element_type=f32)
            bu_s[d] = (bbcc[:, 0:DI].reshape(CH, NS, DI) * u3)
            cb_s[d] = bbcc[:, DI:2 * DI].reshape(CH, NS, DI)

    hs = [h_s[i] for i in range(4)]
    for s in range(CH):
        rows = (s, CH - 1 - s, s, CH - 1 - s)
        for d in range(4):
            t = rows[d]
            hs[d] = da_s[d, t] * hs[d] + bu_s[d, t]
            ha_s[d, t] = hs[d]
    for d in range(4):
        h_s[d] = hs[d]

    dsks = [dsk_ref[0, 0:1, :], dsk_ref[0, 1:2, :],
            dsk_ref[1, 0:1, :], dsk_ref[1, 1:2, :]]
    yrefs = (y0f_ref, y0b_ref, y1f_ref, y1b_ref)
    pt = pt_ref[...]
    for d in range(4):
        prod = (ha_s[d] * cb_s[d]).reshape(CH * NS, DI)
        y2 = jax.lax.dot_general(pt, prod, (((1,), (0,)), ((), ())),
                                 preferred_element_type=f32)
        yv = y2 + dsks[d] * xv4[d]
        if d < 2:
            yrefs[d][...] = yv
        else:
            yrefs[d][:, 0:1, 0:1, :] = yv[0:H].reshape(H, 1, 1, DI)
            yrefs[d][:, 1:2, 0:1, :] = yv[H:CH].reshape(H, 1, 1, DI)


def _fin_kernel(p0_ref, p1_ref, p2_ref, p3_ref, gam_ref, bet_ref, wout_ref,
                o_ref):
    y = p0_ref[...] + p1_ref[...] + p2_ref[...] + p3_ref[...]
    mu = jnp.mean(y, axis=0, keepdims=True)
    d = y - mu
    var = jnp.mean(d * d, axis=0, keepdims=True)
    yn = d * jax.lax.rsqrt(var + 1e-5)
    z = yn * gam_ref[...] + bet_ref[...]
    o_ref[...] = jnp.dot(z, wout_ref[...], preferred_element_type=jnp.float32)


def kernel(x, rep, in_proj_w, sconv_w, sconv_b, sconv_mod_w, sconv_mod_b,
           x_proj_weight, dt_projs_weight, dt_projs_bias, A_logs_w, A_logs_b,
           Ds_w, Ds_b, sain_gamma_w, sain_gamma_b, sain_beta_w, sain_beta_b,
           out_proj_w):
    f32 = jnp.float32
    sds = jax.ShapeDtypeStruct
    xflat = x.reshape(L, DM)
    repf = rep.reshape(REP, L).T

    xc, arow, dskrow, gam, bet = pl.pallas_call(
        _pre_kernel,
        out_shape=[sds((L, DI), f32), sds((1, K * DI * NS), f32),
                   sds((1, K * DI), f32), sds((1, DI), f32), sds((1, DI), f32)],
        scratch_shapes=[pltpu.VMEM((H + 2, W + 2, DI), f32),
                        pltpu.VMEM((H + 2, W, DI), f32),
                        pltpu.VMEM((H + 2, W, DI), f32)],
        compiler_params=pltpu.CompilerParams(vmem_limit_bytes=56 * 2**20),
        name="ss2d_pre",
    )(xflat, repf, in_proj_w.T, sconv_w.reshape(DI, 9).T,
      sconv_b[None, :], sconv_mod_w.T, sconv_mod_b[None, :],
      A_logs_w.T, A_logs_b[None, :], Ds_w.T, Ds_b[None, :],
      sain_gamma_w.T, sain_gamma_b[None, :], sain_beta_w.T,
      sain_beta_b[None, :])

    xc4 = xc.reshape(H, W, 1, DI)
    A_T = arow.reshape(K, DI, NS).transpose(0, 2, 1)
    A_pair = jnp.stack([jnp.stack([A_T[0], A_T[2]]),
                        jnp.stack([A_T[1], A_T[3]])])
    dsk = dskrow.reshape(K, DI)
    dsk_pair = jnp.stack([jnp.stack([dsk[0], dsk[2]]),
                          jnp.stack([dsk[1], dsk[3]])])
    wx = x_proj_weight
    w76 = jnp.stack([jnp.concatenate([wx[0], wx[2]], axis=0),
                     jnp.concatenate([wx[1], wx[3]], axis=0)])
    log2e = jnp.float32(1.4426950408889634)
    z6 = jnp.zeros((RK, DI), f32)
    dtw = dt_projs_weight
    dtw_pair = jnp.stack([
        jnp.concatenate([jnp.concatenate([dtw[0].T, z6], axis=1),
                         jnp.concatenate([z6, dtw[2].T], axis=1)], axis=0),
        jnp.concatenate([jnp.concatenate([dtw[1].T, z6], axis=1),
                         jnp.concatenate([z6, dtw[3].T], axis=1)], axis=0),
    ]) * log2e
    dtb = dt_projs_bias
    dtb_pair = jnp.stack([
        jnp.concatenate([dtb[0], dtb[2]])[None, :],
        jnp.concatenate([dtb[1], dtb[3]])[None, :],
    ]) * log2e

    blk = lambda im: pl.BlockSpec((CH, DI), im)
    blk4 = lambda im: pl.BlockSpec((H, CH // H, 1, DI), im)
    fwd = lambda c: (c, 0)
    bwd = lambda c: (NCH - 1 - c, 0)
    fwd4 = lambda c: (0, c, 0, 0)
    bwd4 = lambda c: (0, NCH - 1 - c, 0, 0)
    ri = jnp.arange(CH * NS, dtype=jnp.int32)[:, None] % NS
    ci = jnp.arange(2 * NS, dtype=jnp.int32)[None, :] % NS
    sel = (ri == ci).astype(f32)
    br = jnp.arange(2 * NS)[:, None] < NS
    bc = jnp.arange(2 * DI)[None, :] < DI
    bones = jnp.where(br & bc, jnp.float32(0.6931471805599453), 0.0) \
        + ((~br) & (~bc)).astype(f32)
    y0f, y0b, y1f, y1b = pl.pallas_call(
        _scan_kernel,
        grid=(NCH,),
        in_specs=[
            blk(fwd), blk(bwd), blk4(fwd4), blk4(bwd4),
            pl.BlockSpec((2, 76, DI), lambda c: (0, 0, 0)),
            pl.BlockSpec((2, 2 * RK, 2 * DI), lambda c: (0, 0, 0)),
            pl.BlockSpec((2, 1, 2 * DI), lambda c: (0, 0, 0)),
            pl.BlockSpec((2, 2, NS, DI), lambda c: (0, 0, 0, 0)),
            pl.BlockSpec((2, 2, DI), lambda c: (0, 0, 0)),
            pl.BlockSpec((CH, CH * NS), lambda c: (0, 0)),
            pl.BlockSpec((CH * NS, 2 * NS), lambda c: (0, 0)),
            pl.BlockSpec((2 * NS, 2 * DI), lambda c: (0, 0)),
        ],
        out_specs=[blk(fwd), blk(bwd), blk4(fwd4), blk4(bwd4)],
        out_shape=[sds((L, DI), f32), sds((L, DI), f32),
                   sds((H, W, 1, DI), f32), sds((H, W, 1, DI), f32)],
        scratch_shapes=[
            pltpu.VMEM((4, CH, NS, DI), f32),
            pltpu.VMEM((4, CH, NS, DI), f32),
            pltpu.VMEM((4, CH, NS, DI), f32),
            pltpu.VMEM((4, CH, NS, DI), f32),
            pltpu.VMEM((4, NS, DI), f32),
        ],
        compiler_params=pltpu.CompilerParams(
            dimension_semantics=(pltpu.GridDimensionSemantics.ARBITRARY,),
            vmem_limit_bytes=48 * 2**20,
        ),
        name="ss2d_scan",
    )(xc, xc, xc4, xc4, w76, dtw_pair, dtb_pair, A_pair, dsk_pair,
      jnp.repeat(jnp.eye(CH, dtype=f32), NS, axis=1), sel, bones)

    p2 = y1f.reshape(L, DI)
    p3 = y1b.reshape(L, DI)
    out = pl.pallas_call(
        _fin_kernel,
        out_shape=sds((L, DM), f32),
        compiler_params=pltpu.CompilerParams(vmem_limit_bytes=56 * 2**20),
        name="ss2d_fin",
    )(y0f, y0b, p2, p3, gam, bet, out_proj_w.T)
    return out.reshape(1, H, W, DM)
